# Initial kernel scaffold; baseline (speedup 1.0000x reference)
import jax
import jax.numpy as jnp
from jax.experimental import pallas as pl


def kernel(node_rep, edge_index, edge_attr, We1, be1, We2, be2, Wn1, bn1, Wn2, bn2):
    raise NotImplementedError("write your pallas kernel here")


# speedup vs baseline: 3.8940x; 3.8940x over previous
import functools

import jax
import jax.numpy as jnp
from jax import lax
from jax.experimental import pallas as pl
from jax.experimental.pallas import tpu as pltpu
from jax.experimental.pallas import tpu_sc as plsc

REP = 128
HID = 2 * REP
N_NODES = 10000
N_EDGES = 320000

NC = 2
NS = 16
NW = NC * NS
EPW = N_EDGES // NW
CHUNK = 80
NCHUNK = EPW // CHUNK
N_NODES_PAD = 10240
NPW = N_NODES_PAD // NS

@functools.cache
def _build_sc_kernels():
    mesh = plsc.VectorSubcoreMesh(core_axis_name="c", subcore_axis_name="s")

    @functools.partial(
        pl.kernel,
        mesh=mesh,
        out_type=(
            jax.ShapeDtypeStruct((N_EDGES, REP), jnp.float32),
            jax.ShapeDtypeStruct((N_EDGES, REP), jnp.float32),
        ),
        scratch_types=[
            pltpu.VMEM((NCHUNK, CHUNK), jnp.int32),
            pltpu.VMEM((NCHUNK, CHUNK), jnp.int32),
            pltpu.VMEM((CHUNK, REP), jnp.float32),
            pltpu.VMEM((CHUNK, REP), jnp.float32),
            pltpu.SemaphoreType.DMA,
            pltpu.SemaphoreType.DMA,
        ],
    )
    def sc_gather(table, src_r, dst_r, gsrc, gdst,
                  idx_s, idx_d, rows_s, rows_d, sem_s, sem_d):
        c = lax.axis_index("c")
        s = lax.axis_index("s")
        base = (c * NS + s) * EPW
        pltpu.sync_copy(src_r.at[c, s], idx_s)
        pltpu.sync_copy(dst_r.at[c, s], idx_d)

        def body(i, carry):
            cp_s = pltpu.async_copy(table.at[idx_s.at[i]], rows_s, sem_s)
            cp_d = pltpu.async_copy(table.at[idx_d.at[i]], rows_d, sem_d)
            cp_s.wait()
            cp_d.wait()
            off = base + i * CHUNK
            pltpu.sync_copy(rows_s, gsrc.at[pl.ds(off, CHUNK)])
            pltpu.sync_copy(rows_d, gdst.at[pl.ds(off, CHUNK)])
            return carry

        lax.fori_loop(0, NCHUNK, body, 0)

    @functools.partial(
        pl.kernel,
        mesh=mesh,
        out_type=jax.ShapeDtypeStruct((NC, N_NODES_PAD, REP), jnp.float32),
        scratch_types=[
            pltpu.VMEM((NCHUNK, CHUNK), jnp.int32),
            pltpu.VMEM((NCHUNK, CHUNK), jnp.int32),
            pltpu.VMEM((CHUNK, REP), jnp.float32),
            pltpu.VMEM_SHARED((N_NODES_PAD, REP), jnp.float32),
        ],
    )
    def sc_scatter(eo_r, src_r, dst_r, zeros, out, idx_s, idx_d, rows, acc):
        c = lax.axis_index("c")
        s = lax.axis_index("s")
        pltpu.sync_copy(zeros.at[pl.ds(s * NPW, NPW)], acc.at[pl.ds(s * NPW, NPW)])
        pltpu.sync_copy(src_r.at[c, s], idx_s)
        pltpu.sync_copy(dst_r.at[c, s], idx_d)
        plsc.subcore_barrier()

        def body(i, carry):
            pltpu.sync_copy(eo_r.at[c, s, i], rows)
            pltpu.sync_copy(rows, acc.at[idx_s.at[i]], add=True)
            pltpu.sync_copy(rows, acc.at[idx_d.at[i]], add=True)
            return carry

        lax.fori_loop(0, NCHUNK, body, 0)
        plsc.subcore_barrier()
        pltpu.sync_copy(acc.at[pl.ds(s * NPW, NPW)],
                        out.at[c].at[pl.ds(s * NPW, NPW)])

    return sc_gather, sc_scatter


def _edge_body(ea, gs, gd, w1a, w1b, w1c, b1, w2, b2, out):
    h = (jnp.dot(ea[...], w1a[...], preferred_element_type=jnp.float32)
         + jnp.dot(gs[...], w1b[...], preferred_element_type=jnp.float32)
         + jnp.dot(gd[...], w1c[...], preferred_element_type=jnp.float32)
         + b1[...])
    h = jnp.maximum(h, 0.0)
    out[...] = jnp.dot(h, w2[...], preferred_element_type=jnp.float32) + b2[...]


def _node_body(nr, p0, p1, w1a, w1b, b1, w2, b2, out):
    e2n = p0[...] + p1[...]
    g = (jnp.dot(nr[...], w1a[...], preferred_element_type=jnp.float32)
         + jnp.dot(e2n, w1b[...], preferred_element_type=jnp.float32)
         + b1[...])
    g = jnp.maximum(g, 0.0)
    out[...] = jnp.dot(g, w2[...], preferred_element_type=jnp.float32) + b2[...]


_BE = 2000
_BN = 1000


def _full(shape):
    return pl.BlockSpec(shape, lambda i: (0, 0))


def _edge_mlp(edge_attr, gsrc, gdst, w1a, w1b, w1c, b1, w2, b2):
    row = pl.BlockSpec((_BE, REP), lambda i: (i, 0))
    return pl.pallas_call(
        _edge_body,
        grid=(N_EDGES // _BE,),
        in_specs=[row, row, row,
                  _full((REP, HID)), _full((REP, HID)), _full((REP, HID)),
                  _full((1, HID)), _full((HID, REP)), _full((1, REP))],
        out_specs=row,
        out_shape=jax.ShapeDtypeStruct((N_EDGES, REP), jnp.float32),
    )(edge_attr, gsrc, gdst, w1a, w1b, w1c, b1, w2, b2)


def _node_mlp(node_rep, p0, p1, w1a, w1b, b1, w2, b2):
    row = pl.BlockSpec((_BN, REP), lambda i: (i, 0))
    return pl.pallas_call(
        _node_body,
        grid=(N_NODES // _BN,),
        in_specs=[row, row, row,
                  _full((REP, HID)), _full((REP, HID)),
                  _full((1, HID)), _full((HID, REP)), _full((1, REP))],
        out_specs=row,
        out_shape=jax.ShapeDtypeStruct((N_NODES, REP), jnp.float32),
    )(node_rep, p0, p1, w1a, w1b, b1, w2, b2)


def kernel(node_rep, edge_index, edge_attr, We1, be1, We2, be2, Wn1, bn1, Wn2, bn2):
    src = edge_index[0].astype(jnp.int32)
    dst = edge_index[1].astype(jnp.int32)
    src_r = src.reshape(NC, NS, NCHUNK, CHUNK)
    dst_r = dst.reshape(NC, NS, NCHUNK, CHUNK)

    sc_gather, sc_scatter = _build_sc_kernels()
    gsrc, gdst = sc_gather(node_rep, src_r, dst_r)

    edge_out = _edge_mlp(edge_attr, gsrc, gdst,
                         We1[:REP], We1[REP:2 * REP], We1[2 * REP:],
                         be1.reshape(1, HID), We2, be2.reshape(1, REP))

    eo_r = edge_out.reshape(NC, NS, NCHUNK, CHUNK, REP)
    zeros = jnp.zeros((N_NODES_PAD, REP), jnp.float32)
    partials = sc_scatter(eo_r, src_r, dst_r, zeros)

    node_out = _node_mlp(node_rep, partials[0, :N_NODES], partials[1, :N_NODES],
                         Wn1[:REP], Wn1[REP:],
                         bn1.reshape(1, HID), Wn2, bn2.reshape(1, REP))
    return node_out, edge_out


# speedup vs baseline: 3.8966x; 1.0007x over previous
import functools

import jax
import jax.numpy as jnp
from jax import lax
from jax.experimental import pallas as pl
from jax.experimental.pallas import tpu as pltpu
from jax.experimental.pallas import tpu_sc as plsc

REP = 128
HID = 2 * REP
N_NODES = 10000
N_EDGES = 320000

NC = 2
NS = 16
NW = NC * NS
EPW = N_EDGES // NW
CHUNK = 80
NCHUNK = EPW // CHUNK
N_NODES_PAD = 10240
NPW = N_NODES_PAD // NS

@functools.cache
def _build_sc_kernels():
    mesh = plsc.VectorSubcoreMesh(core_axis_name="c", subcore_axis_name="s")

    @functools.partial(
        pl.kernel,
        mesh=mesh,
        out_type=(
            jax.ShapeDtypeStruct((N_EDGES, REP), jnp.float32),
            jax.ShapeDtypeStruct((N_EDGES, REP), jnp.float32),
        ),
        scratch_types=[
            pltpu.VMEM((NCHUNK, CHUNK), jnp.int32),
            pltpu.VMEM((NCHUNK, CHUNK), jnp.int32),
            pltpu.VMEM((CHUNK, REP), jnp.float32),
            pltpu.VMEM((CHUNK, REP), jnp.float32),
            pltpu.SemaphoreType.DMA,
            pltpu.SemaphoreType.DMA,
        ],
    )
    def sc_gather(table, src_r, dst_r, gsrc, gdst,
                  idx_s, idx_d, rows_s, rows_d, sem_s, sem_d):
        c = lax.axis_index("c")
        s = lax.axis_index("s")
        base = (c * NS + s) * EPW
        pltpu.sync_copy(src_r.at[c, s], idx_s)
        pltpu.sync_copy(dst_r.at[c, s], idx_d)

        def body(i, carry):
            cp_s = pltpu.async_copy(table.at[idx_s.at[i]], rows_s, sem_s)
            cp_d = pltpu.async_copy(table.at[idx_d.at[i]], rows_d, sem_d)
            cp_s.wait()
            cp_d.wait()
            off = base + i * CHUNK
            pltpu.sync_copy(rows_s, gsrc.at[pl.ds(off, CHUNK)])
            pltpu.sync_copy(rows_d, gdst.at[pl.ds(off, CHUNK)])
            return carry

        lax.fori_loop(0, NCHUNK, body, 0)

    @functools.partial(
        pl.kernel,
        mesh=mesh,
        out_type=jax.ShapeDtypeStruct((NC, N_NODES_PAD, REP), jnp.float32),
        scratch_types=[
            pltpu.VMEM((NCHUNK, CHUNK), jnp.int32),
            pltpu.VMEM((NCHUNK, CHUNK), jnp.int32),
            pltpu.VMEM((CHUNK, REP), jnp.float32),
            pltpu.VMEM_SHARED((N_NODES_PAD, REP), jnp.float32),
        ],
    )
    def sc_scatter(eo_r, src_r, dst_r, zeros, out, idx_s, idx_d, rows, acc):
        c = lax.axis_index("c")
        s = lax.axis_index("s")
        pltpu.sync_copy(zeros.at[pl.ds(s * NPW, NPW)], acc.at[pl.ds(s * NPW, NPW)])
        pltpu.sync_copy(src_r.at[c, s], idx_s)
        pltpu.sync_copy(dst_r.at[c, s], idx_d)
        plsc.subcore_barrier()

        def body(i, carry):
            pltpu.sync_copy(eo_r.at[c, s, i], rows)
            pltpu.sync_copy(rows, acc.at[idx_s.at[i]], add=True)
            pltpu.sync_copy(rows, acc.at[idx_d.at[i]], add=True)
            return carry

        lax.fori_loop(0, NCHUNK, body, 0)
        plsc.subcore_barrier()
        pltpu.sync_copy(acc.at[pl.ds(s * NPW, NPW)],
                        out.at[c].at[pl.ds(s * NPW, NPW)])

    return sc_gather, sc_scatter


def _edge_body(ea, gs, gd, w1a, w1b, w1c, b1, w2, b2, out):
    eab = ea[...].astype(jnp.bfloat16)
    h = (jnp.dot(eab, w1a[...], preferred_element_type=jnp.float32)
         + jnp.dot(gs[...].astype(jnp.bfloat16), w1b[...],
                   preferred_element_type=jnp.float32)
         + jnp.dot(gd[...].astype(jnp.bfloat16), w1c[...],
                   preferred_element_type=jnp.float32)
         + b1[...])
    h = jnp.maximum(h, 0.0).astype(jnp.bfloat16)
    out[...] = jnp.dot(h, w2[...], preferred_element_type=jnp.float32) + b2[...]


def _node_body(nr, p0, p1, w1a, w1b, b1, w2, b2, out):
    e2n = p0[...] + p1[...]
    g = (jnp.dot(nr[...], w1a[...], preferred_element_type=jnp.float32)
         + jnp.dot(e2n, w1b[...], preferred_element_type=jnp.float32)
         + b1[...])
    g = jnp.maximum(g, 0.0)
    out[...] = jnp.dot(g, w2[...], preferred_element_type=jnp.float32) + b2[...]


_BE = 2000
_BN = 1000


def _full(shape):
    return pl.BlockSpec(shape, lambda i: (0, 0))


def _edge_mlp(edge_attr, gsrc, gdst, w1a, w1b, w1c, b1, w2, b2):
    row = pl.BlockSpec((_BE, REP), lambda i: (i, 0))
    return pl.pallas_call(
        _edge_body,
        grid=(N_EDGES // _BE,),
        in_specs=[row, row, row,
                  _full((REP, HID)), _full((REP, HID)), _full((REP, HID)),
                  _full((1, HID)), _full((HID, REP)), _full((1, REP))],
        out_specs=row,
        out_shape=jax.ShapeDtypeStruct((N_EDGES, REP), jnp.float32),
    )(edge_attr, gsrc, gdst,
      w1a.astype(jnp.bfloat16), w1b.astype(jnp.bfloat16),
      w1c.astype(jnp.bfloat16), b1, w2.astype(jnp.bfloat16), b2)


def _node_mlp(node_rep, p0, p1, w1a, w1b, b1, w2, b2):
    row = pl.BlockSpec((_BN, REP), lambda i: (i, 0))
    return pl.pallas_call(
        _node_body,
        grid=(N_NODES // _BN,),
        in_specs=[row, row, row,
                  _full((REP, HID)), _full((REP, HID)),
                  _full((1, HID)), _full((HID, REP)), _full((1, REP))],
        out_specs=row,
        out_shape=jax.ShapeDtypeStruct((N_NODES, REP), jnp.float32),
    )(node_rep, p0, p1, w1a, w1b, b1, w2, b2)


def kernel(node_rep, edge_index, edge_attr, We1, be1, We2, be2, Wn1, bn1, Wn2, bn2):
    src = edge_index[0].astype(jnp.int32)
    dst = edge_index[1].astype(jnp.int32)
    src_r = src.reshape(NC, NS, NCHUNK, CHUNK)
    dst_r = dst.reshape(NC, NS, NCHUNK, CHUNK)

    sc_gather, sc_scatter = _build_sc_kernels()
    gsrc, gdst = sc_gather(node_rep, src_r, dst_r)

    edge_out = _edge_mlp(edge_attr, gsrc, gdst,
                         We1[:REP], We1[REP:2 * REP], We1[2 * REP:],
                         be1.reshape(1, HID), We2, be2.reshape(1, REP))

    eo_r = edge_out.reshape(NC, NS, NCHUNK, CHUNK, REP)
    zeros = jnp.zeros((N_NODES_PAD, REP), jnp.float32)
    partials = sc_scatter(eo_r, src_r, dst_r, zeros)

    node_out = _node_mlp(node_rep, partials[0, :N_NODES], partials[1, :N_NODES],
                         Wn1[:REP], Wn1[REP:],
                         bn1.reshape(1, HID), Wn2, bn2.reshape(1, REP))
    return node_out, edge_out


# speedup vs baseline: 4.5509x; 1.1679x over previous
import functools

import jax
import jax.numpy as jnp
from jax import lax
from jax.experimental import pallas as pl
from jax.experimental.pallas import tpu as pltpu
from jax.experimental.pallas import tpu_sc as plsc

REP = 128
HID = 2 * REP
N_NODES = 10000
N_EDGES = 320000

NC = 2
NS = 16
NW = NC * NS
CHUNK = 80

NSLICE = 5
E_SLICE = N_EDGES // NSLICE
EPS = E_SLICE // NW
G_NCHUNK = EPS // CHUNK

EPW = N_EDGES // NW
S_NCHUNK = EPW // CHUNK
N_NODES_PAD = 10240
NPW = N_NODES_PAD // NS


@functools.cache
def _build_sc_kernels():
    mesh = plsc.VectorSubcoreMesh(core_axis_name="c", subcore_axis_name="s")

    @functools.partial(
        pl.kernel,
        mesh=mesh,
        out_type=(
            jax.ShapeDtypeStruct((E_SLICE, REP), jnp.float32),
            jax.ShapeDtypeStruct((E_SLICE, REP), jnp.float32),
        ),
        scratch_types=[
            pltpu.VMEM((G_NCHUNK, CHUNK), jnp.int32),
            pltpu.VMEM((G_NCHUNK, CHUNK), jnp.int32),
            pltpu.VMEM((CHUNK, REP), jnp.float32),
            pltpu.VMEM((CHUNK, REP), jnp.float32),
            pltpu.SemaphoreType.DMA,
            pltpu.SemaphoreType.DMA,
        ],
    )
    def sc_gather(table, src_r, dst_r, gsrc, gdst,
                  idx_s, idx_d, rows_s, rows_d, sem_s, sem_d):
        c = lax.axis_index("c")
        s = lax.axis_index("s")
        base = (c * NS + s) * EPS
        pltpu.sync_copy(src_r.at[c, s], idx_s)
        pltpu.sync_copy(dst_r.at[c, s], idx_d)

        def body(i, carry):
            cp_s = pltpu.async_copy(table.at[idx_s.at[i]], rows_s, sem_s)
            cp_d = pltpu.async_copy(table.at[idx_d.at[i]], rows_d, sem_d)
            cp_s.wait()
            cp_d.wait()
            off = base + i * CHUNK
            pltpu.sync_copy(rows_s, gsrc.at[pl.ds(off, CHUNK)])
            pltpu.sync_copy(rows_d, gdst.at[pl.ds(off, CHUNK)])
            return carry

        lax.fori_loop(0, G_NCHUNK, body, 0)

    @functools.partial(
        pl.kernel,
        mesh=mesh,
        out_type=jax.ShapeDtypeStruct((NC, N_NODES_PAD, REP), jnp.float32),
        scratch_types=[
            pltpu.VMEM((S_NCHUNK, CHUNK), jnp.int32),
            pltpu.VMEM((S_NCHUNK, CHUNK), jnp.int32),
            pltpu.VMEM((CHUNK, REP), jnp.float32),
            pltpu.VMEM_SHARED((N_NODES_PAD, REP), jnp.float32),
        ],
    )
    def sc_scatter(eo_r, src_r, dst_r, zeros, out, idx_s, idx_d, rows, acc):
        c = lax.axis_index("c")
        s = lax.axis_index("s")
        pltpu.sync_copy(zeros.at[pl.ds(s * NPW, NPW)], acc.at[pl.ds(s * NPW, NPW)])
        pltpu.sync_copy(src_r.at[c, s], idx_s)
        pltpu.sync_copy(dst_r.at[c, s], idx_d)
        plsc.subcore_barrier()

        def body(i, carry):
            pltpu.sync_copy(eo_r.at[c, s, i], rows)
            pltpu.sync_copy(rows, acc.at[idx_s.at[i]], add=True)
            pltpu.sync_copy(rows, acc.at[idx_d.at[i]], add=True)
            return carry

        lax.fori_loop(0, S_NCHUNK, body, 0)
        plsc.subcore_barrier()
        pltpu.sync_copy(acc.at[pl.ds(s * NPW, NPW)],
                        out.at[c].at[pl.ds(s * NPW, NPW)])

    return sc_gather, sc_scatter


def _edge_math(ea, gs, gd, w1a, w1b, w1c, b1, w2, b2, out):
    f32 = jnp.float32
    bf = jnp.bfloat16
    h = (jnp.dot(ea[...].astype(bf), w1a[...], preferred_element_type=f32)
         + jnp.dot(gs[...].astype(bf), w1b[...], preferred_element_type=f32)
         + jnp.dot(gd[...].astype(bf), w1c[...], preferred_element_type=f32)
         + b1[...])
    h = jnp.maximum(h, 0.0).astype(bf)
    out[...] = jnp.dot(h, w2[...], preferred_element_type=f32) + b2[...]


def _edge_body_first(ea, gs, gd, w1a, w1b, w1c, b1, w2, b2, out):
    _edge_math(ea, gs, gd, w1a, w1b, w1c, b1, w2, b2, out)


def _edge_body_acc(prev, ea, gs, gd, w1a, w1b, w1c, b1, w2, b2, out):
    del prev
    _edge_math(ea, gs, gd, w1a, w1b, w1c, b1, w2, b2, out)


def _node_body(nr, p0, p1, w1a, w1b, b1, w2, b2, out):
    e2n = p0[...] + p1[...]
    g = (jnp.dot(nr[...], w1a[...], preferred_element_type=jnp.float32)
         + jnp.dot(e2n, w1b[...], preferred_element_type=jnp.float32)
         + b1[...])
    g = jnp.maximum(g, 0.0)
    out[...] = jnp.dot(g, w2[...], preferred_element_type=jnp.float32) + b2[...]


_BE = 2000
NBLK = E_SLICE // _BE
_BN = 1000


def _full(shape):
    return pl.BlockSpec(shape, lambda i: (0, 0))


def _edge_mlp_slice(q, eo_prev, edge_attr, gs, gd, weights):
    row_g = pl.BlockSpec((_BE, REP), lambda i, q=q: (q * NBLK + i, 0))
    row_l = pl.BlockSpec((_BE, REP), lambda i: (i, 0))
    wspecs = [_full((REP, HID)), _full((REP, HID)), _full((REP, HID)),
              _full((1, HID)), _full((HID, REP)), _full((1, REP))]
    if eo_prev is None:
        return pl.pallas_call(
            _edge_body_first,
            grid=(NBLK,),
            in_specs=[row_g, row_l, row_l] + wspecs,
            out_specs=row_g,
            out_shape=jax.ShapeDtypeStruct((N_EDGES, REP), jnp.float32),
        )(edge_attr, gs, gd, *weights)
    return pl.pallas_call(
        _edge_body_acc,
        grid=(NBLK,),
        in_specs=[pl.BlockSpec(memory_space=pl.ANY), row_g, row_l, row_l] + wspecs,
        out_specs=row_g,
        out_shape=jax.ShapeDtypeStruct((N_EDGES, REP), jnp.float32),
        input_output_aliases={0: 0},
    )(eo_prev, edge_attr, gs, gd, *weights)


def _node_mlp(node_rep, p0, p1, w1a, w1b, b1, w2, b2):
    row = pl.BlockSpec((_BN, REP), lambda i: (i, 0))
    return pl.pallas_call(
        _node_body,
        grid=(N_NODES // _BN,),
        in_specs=[row, row, row,
                  _full((REP, HID)), _full((REP, HID)),
                  _full((1, HID)), _full((HID, REP)), _full((1, REP))],
        out_specs=row,
        out_shape=jax.ShapeDtypeStruct((N_NODES, REP), jnp.float32),
    )(node_rep, p0, p1, w1a, w1b, b1, w2, b2)


def kernel(node_rep, edge_index, edge_attr, We1, be1, We2, be2, Wn1, bn1, Wn2, bn2):
    bf = jnp.bfloat16
    src = edge_index[0].astype(jnp.int32)
    dst = edge_index[1].astype(jnp.int32)
    src_g = src.reshape(NSLICE, NC, NS, G_NCHUNK, CHUNK)
    dst_g = dst.reshape(NSLICE, NC, NS, G_NCHUNK, CHUNK)

    sc_gather, sc_scatter = _build_sc_kernels()

    gathered = [sc_gather(node_rep, src_g[q], dst_g[q]) for q in range(NSLICE)]

    weights = (We1[:REP].astype(bf), We1[REP:2 * REP].astype(bf),
               We1[2 * REP:].astype(bf), be1.reshape(1, HID),
               We2.astype(bf), be2.reshape(1, REP))
    edge_out = None
    for q in range(NSLICE):
        gs, gd = gathered[q]
        edge_out = _edge_mlp_slice(q, edge_out, edge_attr, gs, gd, weights)

    eo_r = edge_out.reshape(NC, NS, S_NCHUNK, CHUNK, REP)
    src_r = src.reshape(NC, NS, S_NCHUNK, CHUNK)
    dst_r = dst.reshape(NC, NS, S_NCHUNK, CHUNK)
    zeros = jnp.zeros((N_NODES_PAD, REP), jnp.float32)
    partials = sc_scatter(eo_r, src_r, dst_r, zeros)

    node_out = _node_mlp(node_rep, partials[0, :N_NODES], partials[1, :N_NODES],
                         Wn1[:REP], Wn1[REP:],
                         bn1.reshape(1, HID), Wn2, bn2.reshape(1, REP))
    return node_out, edge_out


# speedup vs baseline: 5.2507x; 1.1538x over previous
import functools

import jax
import jax.numpy as jnp
from jax import lax
from jax.experimental import pallas as pl
from jax.experimental.pallas import tpu as pltpu
from jax.experimental.pallas import tpu_sc as plsc

REP = 128
HID = 2 * REP
N_NODES = 10000
N_EDGES = 320000

NC = 2
NS = 16
NW = NC * NS
CHUNK = 80

NSLICE = 5
E_SLICE = N_EDGES // NSLICE
EPS = E_SLICE // NW
G_NCHUNK = EPS // CHUNK

EPW = N_EDGES // NW
S_CHUNK = 80
S_NCHUNK = EPW // S_CHUNK
N_NODES_PAD = 10240
NPW = N_NODES_PAD // NS


@functools.cache
def _build_sc_kernels():
    mesh = plsc.VectorSubcoreMesh(core_axis_name="c", subcore_axis_name="s")

    @functools.partial(
        pl.kernel,
        mesh=mesh,
        out_type=(
            jax.ShapeDtypeStruct((E_SLICE, REP), jnp.float32),
            jax.ShapeDtypeStruct((E_SLICE, REP), jnp.float32),
        ),
        scratch_types=[
            pltpu.VMEM((G_NCHUNK, CHUNK), jnp.int32),
            pltpu.VMEM((G_NCHUNK, CHUNK), jnp.int32),
            pltpu.VMEM((2, CHUNK, REP), jnp.float32),
            pltpu.VMEM((2, CHUNK, REP), jnp.float32),
            pltpu.SemaphoreType.DMA((2,)),
            pltpu.SemaphoreType.DMA((2,)),
            pltpu.SemaphoreType.DMA((2,)),
            pltpu.SemaphoreType.DMA((2,)),
        ],
    )
    def sc_gather(table, src_r, dst_r, gsrc, gdst,
                  idx_s, idx_d, rows_s, rows_d, gsem_s, gsem_d, wsem_s, wsem_d):
        c = lax.axis_index("c")
        s = lax.axis_index("s")
        base = (c * NS + s) * EPS
        pltpu.sync_copy(src_r.at[c, s], idx_s)
        pltpu.sync_copy(dst_r.at[c, s], idx_d)

        def start_gather(i, p):
            pltpu.async_copy(table.at[idx_s.at[i]], rows_s.at[p], gsem_s.at[p])
            pltpu.async_copy(table.at[idx_d.at[i]], rows_d.at[p], gsem_d.at[p])

        def wait_gather(i, p):
            pltpu.make_async_copy(table.at[idx_s.at[i]], rows_s.at[p],
                                  gsem_s.at[p]).wait()
            pltpu.make_async_copy(table.at[idx_d.at[i]], rows_d.at[p],
                                  gsem_d.at[p]).wait()

        def wait_write(i, p):
            off = base + i * CHUNK
            pltpu.make_async_copy(rows_s.at[p], gsrc.at[pl.ds(off, CHUNK)],
                                  wsem_s.at[p]).wait()
            pltpu.make_async_copy(rows_d.at[p], gdst.at[pl.ds(off, CHUNK)],
                                  wsem_d.at[p]).wait()

        start_gather(0, 0)

        def body(i, carry):
            p = i % 2

            @pl.when(i + 1 < G_NCHUNK)
            def _():
                @pl.when(i >= 1)
                def _():
                    wait_write(i - 1, 1 - p)
                start_gather(i + 1, 1 - p)

            wait_gather(i, p)
            off = base + i * CHUNK
            pltpu.async_copy(rows_s.at[p], gsrc.at[pl.ds(off, CHUNK)],
                             wsem_s.at[p])
            pltpu.async_copy(rows_d.at[p], gdst.at[pl.ds(off, CHUNK)],
                             wsem_d.at[p])
            return carry

        lax.fori_loop(0, G_NCHUNK, body, 0)
        wait_write(G_NCHUNK - 2, G_NCHUNK % 2)
        wait_write(G_NCHUNK - 1, (G_NCHUNK - 1) % 2)

    @functools.partial(
        pl.kernel,
        mesh=mesh,
        out_type=jax.ShapeDtypeStruct((NC, N_NODES_PAD, REP), jnp.float32),
        scratch_types=[
            pltpu.VMEM((2, S_CHUNK), jnp.int32),
            pltpu.VMEM((2, S_CHUNK), jnp.int32),
            pltpu.VMEM((2, S_CHUNK, REP), jnp.float32),
            pltpu.VMEM_SHARED((N_NODES_PAD, REP), jnp.float32),
            pltpu.SemaphoreType.DMA((2,)),
            pltpu.SemaphoreType.DMA((2,)),
            pltpu.SemaphoreType.DMA((2,)),
        ],
    )
    def sc_scatter(eo_r, src_r, dst_r, zeros, out,
                   idx_s, idx_d, rows, acc, rsem, isem_s, isem_d):
        c = lax.axis_index("c")
        s = lax.axis_index("s")
        pltpu.sync_copy(zeros.at[pl.ds(s * NPW, NPW)], acc.at[pl.ds(s * NPW, NPW)])
        plsc.subcore_barrier()

        def start_chunk(i, p):
            pltpu.async_copy(src_r.at[c, s, i], idx_s.at[p], isem_s.at[p])
            pltpu.async_copy(dst_r.at[c, s, i], idx_d.at[p], isem_d.at[p])
            pltpu.async_copy(eo_r.at[c, s, i], rows.at[p], rsem.at[p])

        def wait_chunk(i, p):
            pltpu.make_async_copy(src_r.at[c, s, i], idx_s.at[p],
                                  isem_s.at[p]).wait()
            pltpu.make_async_copy(dst_r.at[c, s, i], idx_d.at[p],
                                  isem_d.at[p]).wait()
            pltpu.make_async_copy(eo_r.at[c, s, i], rows.at[p],
                                  rsem.at[p]).wait()

        start_chunk(0, 0)

        def body(i, carry):
            p = i % 2

            @pl.when(i + 1 < S_NCHUNK)
            def _():
                start_chunk(i + 1, 1 - p)

            wait_chunk(i, p)
            pltpu.sync_copy(rows.at[p], acc.at[idx_s.at[p]], add=True)
            pltpu.sync_copy(rows.at[p], acc.at[idx_d.at[p]], add=True)
            return carry

        lax.fori_loop(0, S_NCHUNK, body, 0)
        plsc.subcore_barrier()
        pltpu.sync_copy(acc.at[pl.ds(s * NPW, NPW)],
                        out.at[c].at[pl.ds(s * NPW, NPW)])

    return sc_gather, sc_scatter


def _edge_math(ea, gs, gd, w1a, w1b, w1c, b1, w2, b2, out):
    f32 = jnp.float32
    bf = jnp.bfloat16
    h = (jnp.dot(ea[...].astype(bf), w1a[...], preferred_element_type=f32)
         + jnp.dot(gs[...].astype(bf), w1b[...], preferred_element_type=f32)
         + jnp.dot(gd[...].astype(bf), w1c[...], preferred_element_type=f32)
         + b1[...])
    h = jnp.maximum(h, 0.0).astype(bf)
    out[...] = jnp.dot(h, w2[...], preferred_element_type=f32) + b2[...]


def _edge_body_first(ea, gs, gd, w1a, w1b, w1c, b1, w2, b2, out):
    _edge_math(ea, gs, gd, w1a, w1b, w1c, b1, w2, b2, out)


def _edge_body_acc(prev, ea, gs, gd, w1a, w1b, w1c, b1, w2, b2, out):
    del prev
    _edge_math(ea, gs, gd, w1a, w1b, w1c, b1, w2, b2, out)


def _node_body(nr, p0, p1, w1a, w1b, b1, w2, b2, out):
    e2n = p0[...] + p1[...]
    g = (jnp.dot(nr[...], w1a[...], preferred_element_type=jnp.float32)
         + jnp.dot(e2n, w1b[...], preferred_element_type=jnp.float32)
         + b1[...])
    g = jnp.maximum(g, 0.0)
    out[...] = jnp.dot(g, w2[...], preferred_element_type=jnp.float32) + b2[...]


_BE = 2000
NBLK = E_SLICE // _BE
_BN = 1000


def _full(shape):
    return pl.BlockSpec(shape, lambda i: (0, 0))


def _edge_mlp_slice(q, eo_prev, edge_attr, gs, gd, weights):
    row_g = pl.BlockSpec((_BE, REP), lambda i, q=q: (q * NBLK + i, 0))
    row_l = pl.BlockSpec((_BE, REP), lambda i: (i, 0))
    wspecs = [_full((REP, HID)), _full((REP, HID)), _full((REP, HID)),
              _full((1, HID)), _full((HID, REP)), _full((1, REP))]
    if eo_prev is None:
        return pl.pallas_call(
            _edge_body_first,
            grid=(NBLK,),
            in_specs=[row_g, row_l, row_l] + wspecs,
            out_specs=row_g,
            out_shape=jax.ShapeDtypeStruct((N_EDGES, REP), jnp.float32),
        )(edge_attr, gs, gd, *weights)
    return pl.pallas_call(
        _edge_body_acc,
        grid=(NBLK,),
        in_specs=[pl.BlockSpec(memory_space=pl.ANY), row_g, row_l, row_l] + wspecs,
        out_specs=row_g,
        out_shape=jax.ShapeDtypeStruct((N_EDGES, REP), jnp.float32),
        input_output_aliases={0: 0},
    )(eo_prev, edge_attr, gs, gd, *weights)


def _node_mlp(node_rep, p0, p1, w1a, w1b, b1, w2, b2):
    row = pl.BlockSpec((_BN, REP), lambda i: (i, 0))
    return pl.pallas_call(
        _node_body,
        grid=(N_NODES // _BN,),
        in_specs=[row, row, row,
                  _full((REP, HID)), _full((REP, HID)),
                  _full((1, HID)), _full((HID, REP)), _full((1, REP))],
        out_specs=row,
        out_shape=jax.ShapeDtypeStruct((N_NODES, REP), jnp.float32),
    )(node_rep, p0, p1, w1a, w1b, b1, w2, b2)


def kernel(node_rep, edge_index, edge_attr, We1, be1, We2, be2, Wn1, bn1, Wn2, bn2):
    bf = jnp.bfloat16
    src = edge_index[0].astype(jnp.int32)
    dst = edge_index[1].astype(jnp.int32)
    src_g = src.reshape(NSLICE, NC, NS, G_NCHUNK, CHUNK)
    dst_g = dst.reshape(NSLICE, NC, NS, G_NCHUNK, CHUNK)

    sc_gather, sc_scatter = _build_sc_kernels()

    gathered = [sc_gather(node_rep, src_g[q], dst_g[q]) for q in range(NSLICE)]

    weights = (We1[:REP].astype(bf), We1[REP:2 * REP].astype(bf),
               We1[2 * REP:].astype(bf), be1.reshape(1, HID),
               We2.astype(bf), be2.reshape(1, REP))
    edge_out = None
    for q in range(NSLICE):
        gs, gd = gathered[q]
        edge_out = _edge_mlp_slice(q, edge_out, edge_attr, gs, gd, weights)

    eo_r = edge_out.reshape(NC, NS, S_NCHUNK, S_CHUNK, REP)
    src_r = src.reshape(NC, NS, S_NCHUNK, S_CHUNK)
    dst_r = dst.reshape(NC, NS, S_NCHUNK, S_CHUNK)
    zeros = jnp.zeros((N_NODES_PAD, REP), jnp.float32)
    partials = sc_scatter(eo_r, src_r, dst_r, zeros)

    node_out = _node_mlp(node_rep, partials[0, :N_NODES], partials[1, :N_NODES],
                         Wn1[:REP], Wn1[REP:],
                         bn1.reshape(1, HID), Wn2, bn2.reshape(1, REP))
    return node_out, edge_out
